# baseline (speedup 1.0000x reference)
import functools

import jax
import jax.numpy as jnp
from jax import lax
from jax.experimental import pallas as pl
from jax.experimental.pallas import tpu as pltpu
from jax.experimental.pallas import tpu_sc as plsc

_NUM_ITEMS = 100000
_D = 128
_B = 16384
_NC = 2
_NS = 16
_NW = _NC * _NS
_CH = 128


def _sc_gather(idx, table):
    nrows = idx.shape[0]
    bpw = nrows // _NW
    ch = min(bpw, _CH)
    nch = bpw // ch
    mesh = plsc.VectorSubcoreMesh(
        core_axis_name="c", subcore_axis_name="s",
        num_cores=_NC, num_subcores=_NS)

    @functools.partial(
        pl.kernel,
        out_type=jax.ShapeDtypeStruct((nrows, _D), jnp.float32),
        mesh=mesh,
        scratch_types=[
            pltpu.VMEM((bpw,), jnp.int32),
            pltpu.VMEM((bpw, _D), jnp.float32),
            pltpu.SemaphoreType.DMA,
            pltpu.SemaphoreType.DMA,
        ],
    )
    def gather_kernel(idx_hbm, table_hbm, out_hbm, idx_v, rows_v, sem, wsem):
        wid = lax.axis_index("s") * _NC + lax.axis_index("c")
        base = wid * bpw
        pltpu.sync_copy(idx_hbm.at[pl.ds(base, bpw)], idx_v)
        copies = [
            pltpu.async_copy(
                table_hbm.at[idx_v.at[pl.ds(j * ch, ch)]],
                rows_v.at[pl.ds(j * ch, ch), :],
                sem,
            )
            for j in range(nch)
        ]
        writes = []
        for j, c in enumerate(copies):
            c.wait()
            writes.append(pltpu.async_copy(
                rows_v.at[pl.ds(j * ch, ch), :],
                out_hbm.at[pl.ds(base + j * ch, ch), :],
                wsem,
            ))
        for w in writes:
            w.wait()

    return gather_kernel(idx, table)


def _mlp_body(emb_ref, w1_ref, b1_ref, w2_ref, b2_ref, out_ref):
    x = emb_ref[...]
    h = jnp.dot(x, w1_ref[...], preferred_element_type=jnp.float32)
    h = jnp.maximum(h + b1_ref[...], 0.0)
    o = jnp.dot(h, w2_ref[...], preferred_element_type=jnp.float32)
    o = o + b2_ref[...]
    nsq = jnp.sum(o * o, axis=1, keepdims=True)
    out_ref[...] = o * lax.rsqrt(jnp.maximum(nsq, 1e-24))


def _tc_mlp(emb, w1, b1, w2, b2, block_b=2048):
    nrows = emb.shape[0]
    grid = (nrows // block_b,)
    return pl.pallas_call(
        _mlp_body,
        grid=grid,
        in_specs=[
            pl.BlockSpec((block_b, _D), lambda i: (i, 0)),
            pl.BlockSpec((_D, 2 * _D), lambda i: (0, 0)),
            pl.BlockSpec((1, 2 * _D), lambda i: (0, 0)),
            pl.BlockSpec((2 * _D, _D), lambda i: (0, 0)),
            pl.BlockSpec((1, _D), lambda i: (0, 0)),
        ],
        out_specs=pl.BlockSpec((block_b, _D), lambda i: (i, 0)),
        out_shape=jax.ShapeDtypeStruct((nrows, _D), jnp.float32),
    )(emb, w1, b1, w2, b2)


def kernel(item_ids, table, W1, b1, W2, b2):
    ids = item_ids.astype(jnp.int32)
    b1r, b2r = b1.reshape(1, -1), b2.reshape(1, -1)
    half = _B // 2
    emb0 = _sc_gather(lax.slice_in_dim(ids, 0, half), table)
    emb1 = _sc_gather(lax.slice_in_dim(ids, half, _B), table)
    out0 = _tc_mlp(emb0, W1, b1r, W2, b2r)
    out1 = _tc_mlp(emb1, W1, b1r, W2, b2r)
    return jnp.concatenate([out0, out1], axis=0)


# speedup vs baseline: 1.1755x; 1.1755x over previous
import functools

import jax
import jax.numpy as jnp
from jax import lax
from jax.experimental import pallas as pl
from jax.experimental.pallas import tpu as pltpu
from jax.experimental.pallas import tpu_sc as plsc

_NUM_ITEMS = 100000
_D = 128
_B = 16384
_NC = 2
_NS = 16
_NW = _NC * _NS
_CH = 128


def _sc_gather(idx, table, row0=0, nrows=None):
    if nrows is None:
        nrows = idx.shape[0]
    bpw = nrows // _NW
    ch = min(bpw, _CH)
    nch = bpw // ch
    mesh = plsc.VectorSubcoreMesh(
        core_axis_name="c", subcore_axis_name="s",
        num_cores=_NC, num_subcores=_NS)

    @functools.partial(
        pl.kernel,
        out_type=jax.ShapeDtypeStruct((nrows, _D), jnp.float32),
        mesh=mesh,
        scratch_types=[
            pltpu.VMEM((bpw,), jnp.int32),
            pltpu.VMEM((bpw, _D), jnp.float32),
            pltpu.SemaphoreType.DMA,
            pltpu.SemaphoreType.DMA,
        ],
    )
    def gather_kernel(idx_hbm, table_hbm, out_hbm, idx_v, rows_v, sem, wsem):
        wid = lax.axis_index("s") * _NC + lax.axis_index("c")
        base = wid * bpw
        pltpu.sync_copy(idx_hbm.at[pl.ds(row0 + base, bpw)], idx_v)
        copies = [
            pltpu.async_copy(
                table_hbm.at[idx_v.at[pl.ds(j * ch, ch)]],
                rows_v.at[pl.ds(j * ch, ch), :],
                sem,
            )
            for j in range(nch)
        ]
        writes = []
        for j, c in enumerate(copies):
            c.wait()
            writes.append(pltpu.async_copy(
                rows_v.at[pl.ds(j * ch, ch), :],
                out_hbm.at[pl.ds(base + j * ch, ch), :],
                wsem,
            ))
        for w in writes:
            w.wait()

    return gather_kernel(idx, table)


def _mlp_body(emb_ref, w1_ref, b1_ref, w2_ref, b2_ref, out_ref):
    x = emb_ref[...]
    h = jnp.dot(x, w1_ref[...], preferred_element_type=jnp.float32)
    h = jnp.maximum(h + b1_ref[...], 0.0)
    o = jnp.dot(h, w2_ref[...], preferred_element_type=jnp.float32)
    o = o + b2_ref[...]
    nsq = jnp.sum(o * o, axis=1, keepdims=True)
    out_ref[...] = o * lax.rsqrt(jnp.maximum(nsq, 1e-24))


def _mlp_body_alias(emb_ref, w1_ref, b1_ref, w2_ref, b2_ref, dest_ref,
                    out_ref):
    del dest_ref
    _mlp_body(emb_ref, w1_ref, b1_ref, w2_ref, b2_ref, out_ref)


def _tc_mlp(emb, w1, b1, w2, b2, block_b=2048, dest=None, out_block0=0,
            out_rows=None):
    nrows = emb.shape[0]
    if out_rows is None:
        out_rows = nrows
    grid = (nrows // block_b,)
    in_specs = [
        pl.BlockSpec((block_b, _D), lambda i: (i, 0)),
        pl.BlockSpec((_D, 2 * _D), lambda i: (0, 0)),
        pl.BlockSpec((1, 2 * _D), lambda i: (0, 0)),
        pl.BlockSpec((2 * _D, _D), lambda i: (0, 0)),
        pl.BlockSpec((1, _D), lambda i: (0, 0)),
    ]
    inputs = [emb, w1, b1, w2, b2]
    kwargs = {}
    body = _mlp_body
    if dest is not None:
        in_specs.append(pl.BlockSpec(memory_space=pl.ANY))
        inputs.append(dest)
        kwargs["input_output_aliases"] = {5: 0}
        body = _mlp_body_alias
    return pl.pallas_call(
        body,
        grid=grid,
        in_specs=in_specs,
        out_specs=pl.BlockSpec((block_b, _D),
                               lambda i: (i + out_block0, 0)),
        out_shape=jax.ShapeDtypeStruct((out_rows, _D), jnp.float32),
        **kwargs,
    )(*inputs)


def kernel(item_ids, table, W1, b1, W2, b2):
    ids = item_ids.astype(jnp.int32)
    b1r, b2r = b1.reshape(1, -1), b2.reshape(1, -1)
    half = _B // 2
    blk = 4096
    emb0 = _sc_gather(ids, table, row0=0, nrows=half)
    emb1 = _sc_gather(ids, table, row0=half, nrows=half)
    dest = _tc_mlp(emb0, W1, b1r, W2, b2r, block_b=blk,
                   out_block0=0, out_rows=_B)
    return _tc_mlp(emb1, W1, b1r, W2, b2r, block_b=blk, dest=dest,
                   out_block0=half // blk, out_rows=_B)


# speedup vs baseline: 1.2473x; 1.0611x over previous
import functools

import jax
import jax.numpy as jnp
from jax import lax
from jax.experimental import pallas as pl
from jax.experimental.pallas import tpu as pltpu
from jax.experimental.pallas import tpu_sc as plsc

_NUM_ITEMS = 100000
_D = 128
_B = 16384
_NC = 2
_NS = 16
_NW = _NC * _NS
_CH = 128


def _sc_gather(idx, table, row0=0, nrows=None):
    if nrows is None:
        nrows = idx.shape[0]
    bpw = nrows // _NW
    ch = min(bpw, _CH)
    nch = bpw // ch
    mesh = plsc.VectorSubcoreMesh(
        core_axis_name="c", subcore_axis_name="s",
        num_cores=_NC, num_subcores=_NS)

    @functools.partial(
        pl.kernel,
        out_type=jax.ShapeDtypeStruct((nrows, _D), jnp.float32),
        mesh=mesh,
        scratch_types=[
            pltpu.VMEM((bpw,), jnp.int32),
            pltpu.VMEM((bpw, _D), jnp.float32),
            pltpu.SemaphoreType.DMA,
            pltpu.SemaphoreType.DMA,
        ],
    )
    def gather_kernel(idx_hbm, table_hbm, out_hbm, idx_v, rows_v, sem, wsem):
        wid = lax.axis_index("s") * _NC + lax.axis_index("c")
        base = wid * bpw
        pltpu.sync_copy(idx_hbm.at[pl.ds(row0 + base, bpw)], idx_v)
        copies = [
            pltpu.async_copy(
                table_hbm.at[idx_v.at[pl.ds(j * ch, ch)]],
                rows_v.at[pl.ds(j * ch, ch), :],
                sem,
            )
            for j in range(nch)
        ]
        writes = []
        for j, c in enumerate(copies):
            c.wait()
            writes.append(pltpu.async_copy(
                rows_v.at[pl.ds(j * ch, ch), :],
                out_hbm.at[pl.ds(base + j * ch, ch), :],
                wsem,
            ))
        for w in writes:
            w.wait()

    return gather_kernel(idx, table)


def _mlp_body(emb_ref, w1_ref, b1_ref, w2_ref, b2_ref, out_ref):
    x = emb_ref[...]
    h = jnp.dot(x, w1_ref[...], preferred_element_type=jnp.float32)
    h = jnp.maximum(h + b1_ref[...], 0.0)
    o = jnp.dot(h, w2_ref[...], preferred_element_type=jnp.float32)
    o = o + b2_ref[...]
    nsq = jnp.sum(o * o, axis=1, keepdims=True)
    out_ref[...] = o * lax.rsqrt(jnp.maximum(nsq, 1e-24))


def _mlp_body_alias(emb_ref, w1_ref, b1_ref, w2_ref, b2_ref, dest_ref,
                    out_ref):
    del dest_ref
    _mlp_body(emb_ref, w1_ref, b1_ref, w2_ref, b2_ref, out_ref)


def _tc_mlp(emb, w1, b1, w2, b2, block_b=2048, dest=None, out_block0=0,
            out_rows=None):
    nrows = emb.shape[0]
    if out_rows is None:
        out_rows = nrows
    grid = (nrows // block_b,)
    in_specs = [
        pl.BlockSpec((block_b, _D), lambda i: (i, 0)),
        pl.BlockSpec((_D, 2 * _D), lambda i: (0, 0)),
        pl.BlockSpec((1, 2 * _D), lambda i: (0, 0)),
        pl.BlockSpec((2 * _D, _D), lambda i: (0, 0)),
        pl.BlockSpec((1, _D), lambda i: (0, 0)),
    ]
    inputs = [emb, w1, b1, w2, b2]
    kwargs = {}
    body = _mlp_body
    if dest is not None:
        in_specs.append(pl.BlockSpec(memory_space=pl.ANY))
        inputs.append(dest)
        kwargs["input_output_aliases"] = {5: 0}
        body = _mlp_body_alias
    return pl.pallas_call(
        body,
        grid=grid,
        in_specs=in_specs,
        out_specs=pl.BlockSpec((block_b, _D),
                               lambda i: (i + out_block0, 0)),
        out_shape=jax.ShapeDtypeStruct((out_rows, _D), jnp.float32),
        **kwargs,
    )(*inputs)


def kernel(item_ids, table, W1, b1, W2, b2):
    ids = item_ids.astype(jnp.int32)
    emb = _sc_gather(ids, table)
    return _tc_mlp(emb, W1, b1.reshape(1, -1), W2, b2.reshape(1, -1),
                   block_b=8192)
